# baseline (speedup 1.0000x reference)
import jax
import jax.numpy as jnp
from jax import lax
from jax.experimental import pallas as pl
from jax.experimental.pallas import tpu as pltpu
from jax.experimental.pallas import tpu_sc as plsc

NUM_CORES = 2
NUM_SUBCORES = 16
LANES = 16

CHUNK = 80


def _gcn_sc(xa, xb, src2, dst2, w2, zrows, n_pad, dh, chunks, num_layers):
    rows_per_tile = n_pad // NUM_SUBCORES
    groups = CHUNK // LANES
    nf32 = 2 * num_layers
    nbf = 2 * (num_layers - 1)

    def body(xa_hbm, xb_hbm, src_hbm, dst_hbm, w_hbm, zrows_hbm, *rest):
        outs = rest[:nf32]
        tbls = rest[nf32:nf32 + nbf]
        (srcb, dstb, wb, dstl0, dstl1, rows0, rows1, scaled0, scaled1, acc,
         sem_g0, sem_g1, sem_s0, sem_s1) = rest[nf32 + nbf:]
        cid = lax.axis_index("c")
        tid = lax.axis_index("s")

        pltpu.sync_copy(src_hbm.at[tid], srcb)
        pltpu.sync_copy(dst_hbm.at[tid], dstb)
        pltpu.sync_copy(w_hbm.at[tid], wb)

        my_off = pl.multiple_of(tid * rows_per_tile, 8)

        def scale_chunk(rows_r, scaled_r, dstl_r, i):
            @plsc.parallel_loop(0, groups, 1, unroll=1)
            def _(g):
                off = pl.multiple_of(i * CHUNK + g * LANES, LANES)
                wv16 = wb[pl.ds(off, LANES)]
                dstl_r[pl.ds(g * LANES, LANES)] = dstb[pl.ds(off, LANES)]
                def widen_f16(t):
                    s = t & jnp.int32(-2147483648)
                    em = (t & jnp.int32(0x7FFF0000)) >> 3
                    return plsc.bitcast(s | (em + jnp.int32(112 << 23)),
                                        jnp.float32)

                for l in range(LANES):
                    wv = jnp.full((LANES,), wv16[l])
                    e = g * LANES + l
                    for g2 in range(dh // (2 * LANES)):
                        v = rows_r[e, pl.ds(g2 * LANES, LANES)]
                        lo = widen_f16(v << 16)
                        hi = widen_f16(v & jnp.int32(-65536))
                        scaled_r[e, pl.ds(g2 * 2 * LANES, LANES)] = lo * wv
                        scaled_r[e, pl.ds(g2 * 2 * LANES + LANES, LANES)] = hi * wv

        def run_layers(x_tab, houts, tbs):
            def layer_body(lay, _):
                tabs = [x_tab] + list(tbs)

                def gather_into(i, rows_r, sem):
                    for l2, tab in enumerate(tabs):
                        @pl.when(lay == l2)
                        def _(tab=tab):
                            pltpu.async_copy(tab.at[srcb.at[i]], rows_r, sem)

                def wait_gather(i, rows_r, sem):
                    for l2, tab in enumerate(tabs):
                        @pl.when(lay == l2)
                        def _(tab=tab):
                            pltpu.make_async_copy(tab.at[srcb.at[i]], rows_r, sem).wait()

                pltpu.sync_copy(zrows_hbm, acc.at[pl.ds(my_off, rows_per_tile)])
                gather_into(0, rows0, sem_g0)
                gather_into(1, rows1, sem_g1)
                plsc.subcore_barrier()

                def half_step(j, i, rows_r, scaled_r, dstl_r, sem_g, sem_s):
                    wait_gather(i, rows_r, sem_g)

                    @pl.when(j > 0)
                    def _():
                        pltpu.make_async_copy(scaled_r, acc.at[dstl_r], sem_s).wait()

                    scale_chunk(rows_r, scaled_r, dstl_r, i)

                    @pl.when(i + 2 < chunks)
                    def _():
                        gather_into(i + 2, rows_r, sem_g)

                    pltpu.async_copy(scaled_r, acc.at[dstl_r], sem_s, add=True)

                def pair_body(j, _):
                    half_step(j, 2 * j, rows0, scaled0, dstl0, sem_g0, sem_s0)
                    half_step(j, 2 * j + 1, rows1, scaled1, dstl1, sem_g1, sem_s1)
                    return 0

                lax.fori_loop(0, chunks // 2, pair_body, 0, unroll=False)
                pltpu.make_async_copy(scaled0, acc.at[dstl0], sem_s0).wait()
                pltpu.make_async_copy(scaled1, acc.at[dstl1], sem_s1).wait()
                plsc.subcore_barrier()

                for l2, hout in enumerate(houts):
                    @pl.when(lay == l2)
                    def _(l2=l2, hout=hout):
                        pltpu.sync_copy(acc.at[pl.ds(my_off, rows_per_tile)],
                                        hout.at[pl.ds(my_off, rows_per_tile)])
                        if l2 >= num_layers - 1:
                            return

                        def conv_span(poff, nrows, tb):
                            pltpu.sync_copy(acc.at[pl.ds(poff, nrows)],
                                            scaled0.at[pl.ds(0, nrows)])

                            def narrow_f16(f):
                                bb = plsc.bitcast(f, jnp.int32)
                                s = (bb >> 16) & jnp.int32(0x8000)
                                em = bb & jnp.int32(0x7FFFFFFF)
                                r_ = (em - jnp.int32(112 << 23) + jnp.int32(0x0FFF)
                                      + ((em >> 13) & 1)) >> 13
                                r_ = jnp.minimum(jnp.maximum(r_, 0),
                                                 jnp.int32(0x7BFF))
                                return s | r_

                            def conv_row(r, _3):
                                for g2 in range(dh // (2 * LANES)):
                                    a = narrow_f16(scaled0[r, pl.ds(g2 * 2 * LANES, LANES)])
                                    b = narrow_f16(scaled0[r, pl.ds(g2 * 2 * LANES + LANES, LANES)])
                                    rows0[r, pl.ds(g2 * LANES, LANES)] = a | (b << 16)
                                return 0

                            lax.fori_loop(0, nrows, conv_row, 0, unroll=False)
                            pltpu.sync_copy(rows0.at[pl.ds(0, nrows)],
                                            tb.at[pl.ds(poff, nrows)])

                        def conv_piece(p, _2, tb=tbs[l2]):
                            conv_span(pl.multiple_of(my_off + p * CHUNK, 8), CHUNK, tb)
                            return 0

                        lax.fori_loop(0, rows_per_tile // CHUNK, conv_piece, 0,
                                      unroll=False)
                        rem = rows_per_tile % CHUNK
                        if rem:
                            conv_span(
                                pl.multiple_of(
                                    my_off + (rows_per_tile // CHUNK) * CHUNK, 8),
                                rem, tbs[l2])
                plsc.subcore_barrier()
                return 0

            lax.fori_loop(0, num_layers, layer_body, 0, unroll=False)

        @pl.when(cid == 0)
        def _():
            run_layers(xa_hbm, [outs[2 * l] for l in range(num_layers)],
                       [tbls[2 * l] for l in range(num_layers - 1)])

        @pl.when(cid == 1)
        def _():
            run_layers(xb_hbm, [outs[2 * l + 1] for l in range(num_layers)],
                       [tbls[2 * l + 1] for l in range(num_layers - 1)])

    mesh = plsc.VectorSubcoreMesh(core_axis_name="c", subcore_axis_name="s")
    fn = pl.kernel(
        body,
        out_type=([jax.ShapeDtypeStruct((n_pad, dh), jnp.float32)] * nf32
                  + [jax.ShapeDtypeStruct((n_pad, dh // 2), jnp.int32)] * nbf),
        mesh=mesh,
        compiler_params=pltpu.CompilerParams(use_tc_tiling_on_sc=False,
                                             needs_layout_passes=False),
        scratch_types=[
            pltpu.VMEM((chunks, CHUNK), jnp.int32),
            pltpu.VMEM((chunks * CHUNK,), jnp.int32),
            pltpu.VMEM((chunks * CHUNK,), jnp.float32),
            pltpu.VMEM((CHUNK,), jnp.int32),
            pltpu.VMEM((CHUNK,), jnp.int32),
            pltpu.VMEM((CHUNK, dh // 2), jnp.int32),
            pltpu.VMEM((CHUNK, dh // 2), jnp.int32),
            pltpu.VMEM((CHUNK, dh), jnp.float32),
            pltpu.VMEM((CHUNK, dh), jnp.float32),
            pltpu.VMEM_SHARED((n_pad, dh), jnp.float32),
            pltpu.SemaphoreType.DMA,
            pltpu.SemaphoreType.DMA,
            pltpu.SemaphoreType.DMA,
            pltpu.SemaphoreType.DMA,
        ],
    )
    return fn(xa, xb, src2, dst2, w2, zrows)[:nf32]


def _mlp_body(xa, xb, h1a, h1b, h2a, h2b, h3a, h3b,
              w1_ref, b1_ref, w2_ref, b2_ref, out_ref):
    dh = xa.shape[1]
    parts = (xa, xb, h1a, h1b, h2a, h2b, h3a, h3b)
    acc = b1_ref[...].astype(jnp.float32)
    for k, p in enumerate(parts):
        acc = acc + jnp.dot(p[...], w1_ref[k * dh:(k + 1) * dh, :],
                            preferred_element_type=jnp.float32)
    hmid = jnp.tanh(acc)
    out_ref[...] = jnp.dot(hmid, w2_ref[...], preferred_element_type=jnp.float32) + b2_ref[...]


def _mlp_tc(parts, W1, b1, W2, b2, n, block_rows=1000):
    d = W2.shape[0]
    dh = parts[0].shape[1]
    grid = (n // block_rows,)
    row_spec = pl.BlockSpec((block_rows, dh), lambda i: (i, 0))
    full = lambda shape: pl.BlockSpec(shape, lambda i: tuple(0 for _ in shape))
    return pl.pallas_call(
        _mlp_body,
        grid=grid,
        in_specs=[row_spec] * 8 + [
            full(W1.shape), full((1, d)), full(W2.shape), full((1, d)),
        ],
        out_specs=pl.BlockSpec((block_rows, d), lambda i: (i, 0)),
        out_shape=jax.ShapeDtypeStruct((n, d), jnp.float32),
    )(*parts, W1, b1.reshape(1, d), W2, b2.reshape(1, d))


def kernel(x, edge_index, edge_weight, W1, b1, W2, b2):
    n, d = x.shape
    e = edge_index.shape[1]
    per_tile = e // NUM_SUBCORES
    chunks = per_tile // CHUNK
    num_layers = (W1.shape[0] // d) - 1
    dh = d // NUM_CORES

    align = 8 * NUM_SUBCORES
    n_pad = ((n + align - 1) // align) * align

    xa = x[:, :dh]
    xb = x[:, dh:]

    def _perm_bf16(m):
        n0 = m.shape[0]
        u = lax.bitcast_convert_type(m.astype(jnp.float16), jnp.uint16)
        u = (u.reshape(n0, dh // (2 * LANES), 2, LANES)
             .transpose(0, 1, 3, 2))
        return lax.bitcast_convert_type(u, jnp.int32).reshape(n0, dh // 2)

    src2 = edge_index[0].reshape(NUM_SUBCORES, chunks, CHUNK)
    dst2 = edge_index[1].reshape(NUM_SUBCORES, per_tile)
    w2 = edge_weight.reshape(NUM_SUBCORES, per_tile)
    zrows = jnp.zeros((n_pad // NUM_SUBCORES, dh), dtype=jnp.float32)

    hs = _gcn_sc(_perm_bf16(xa), _perm_bf16(xb), src2, dst2, w2, zrows,
                 n_pad, dh, chunks, num_layers)
    parts = [xa, xb] + [h[:n] for h in hs]
    return _mlp_tc(parts, W1, b1, W2, b2, n)


# speedup vs baseline: 1.5158x; 1.5158x over previous
import jax
import jax.numpy as jnp
from jax import lax
from jax.experimental import pallas as pl
from jax.experimental.pallas import tpu as pltpu
from jax.experimental.pallas import tpu_sc as plsc

NUM_CORES = 2
NUM_SUBCORES = 16
LANES = 16

CHUNK = 80


def _gcn_sc(xa, xb, src2, dst2, w2, zrows, n_pad, dh, chunks, num_layers):
    rows_per_tile = n_pad // NUM_SUBCORES
    groups = CHUNK // LANES
    nf32 = 2 * num_layers
    nbf = 2 * (num_layers - 1)

    def body(xa_hbm, xb_hbm, src_hbm, dst_hbm, w_hbm, zrows_hbm, *rest):
        outs = rest[:nf32]
        tbls = rest[nf32:nf32 + nbf]
        (srcb, dstb, wb, dstl0, dstl1, rows0, rows1, scaled0, scaled1, acc,
         sem_g0, sem_g1, sem_s0, sem_s1) = rest[nf32 + nbf:]
        cid = lax.axis_index("c")
        tid = lax.axis_index("s")

        pltpu.sync_copy(src_hbm.at[tid], srcb)
        pltpu.sync_copy(dst_hbm.at[tid], dstb)
        pltpu.sync_copy(w_hbm.at[tid], wb)

        my_off = pl.multiple_of(tid * rows_per_tile, 8)

        def scale_chunk(rows_r, scaled_r, dstl_r, i):
            @plsc.parallel_loop(0, groups, 1, unroll=1)
            def _(g):
                off = pl.multiple_of(i * CHUNK + g * LANES, LANES)
                wv16 = wb[pl.ds(off, LANES)]
                dstl_r[pl.ds(g * LANES, LANES)] = dstb[pl.ds(off, LANES)]
                wmask = jnp.int32(0x8FFFE000 - (1 << 32))
                wv16c = wv16 * jnp.float32(2.0 ** 112)
                for l in range(LANES):
                    wv = jnp.full((LANES,), wv16c[l])
                    e = g * LANES + l
                    for g2 in range(dh // (2 * LANES)):
                        v = rows_r[e, pl.ds(g2 * LANES, LANES)]
                        lo = plsc.bitcast(((v << 16) >> 3) & wmask, jnp.float32)
                        hi = plsc.bitcast((v >> 3) & wmask, jnp.float32)
                        scaled_r[e, pl.ds(g2 * 2 * LANES, LANES)] = lo * wv
                        scaled_r[e, pl.ds(g2 * 2 * LANES + LANES, LANES)] = hi * wv

        def run_layers(x_tab, houts, tbs):
            def layer_body(lay, _):
                tabs = [x_tab] + list(tbs)

                def gather_into(i, rows_r, sem):
                    for l2, tab in enumerate(tabs):
                        @pl.when(lay == l2)
                        def _(tab=tab):
                            pltpu.async_copy(tab.at[srcb.at[i]], rows_r, sem)

                def wait_gather(i, rows_r, sem):
                    for l2, tab in enumerate(tabs):
                        @pl.when(lay == l2)
                        def _(tab=tab):
                            pltpu.make_async_copy(tab.at[srcb.at[i]], rows_r, sem).wait()

                pltpu.sync_copy(zrows_hbm, acc.at[pl.ds(my_off, rows_per_tile)])
                gather_into(0, rows0, sem_g0)
                gather_into(1, rows1, sem_g1)
                plsc.subcore_barrier()

                def half_step(j, i, rows_r, scaled_r, dstl_r, sem_g, sem_s):
                    wait_gather(i, rows_r, sem_g)

                    @pl.when(j > 0)
                    def _():
                        pltpu.make_async_copy(scaled_r, acc.at[dstl_r], sem_s).wait()

                    scale_chunk(rows_r, scaled_r, dstl_r, i)

                    @pl.when(i + 2 < chunks)
                    def _():
                        gather_into(i + 2, rows_r, sem_g)

                    pltpu.async_copy(scaled_r, acc.at[dstl_r], sem_s, add=True)

                def pair_body(j, _):
                    half_step(j, 2 * j, rows0, scaled0, dstl0, sem_g0, sem_s0)
                    half_step(j, 2 * j + 1, rows1, scaled1, dstl1, sem_g1, sem_s1)
                    return 0

                lax.fori_loop(0, chunks // 2, pair_body, 0, unroll=False)
                pltpu.make_async_copy(scaled0, acc.at[dstl0], sem_s0).wait()
                pltpu.make_async_copy(scaled1, acc.at[dstl1], sem_s1).wait()
                plsc.subcore_barrier()

                for l2, hout in enumerate(houts):
                    @pl.when(lay == l2)
                    def _(l2=l2, hout=hout):
                        pltpu.sync_copy(acc.at[pl.ds(my_off, rows_per_tile)],
                                        hout.at[pl.ds(my_off, rows_per_tile)])
                        if l2 >= num_layers - 1:
                            return

                        def conv_span(poff, nrows, tb):
                            pltpu.sync_copy(acc.at[pl.ds(poff, nrows)],
                                            scaled0.at[pl.ds(0, nrows)])

                            def narrow_f16(f):
                                bb = plsc.bitcast(f, jnp.int32)
                                s = (bb >> 16) & jnp.int32(0x8000)
                                em = bb & jnp.int32(0x7FFFFFFF)
                                r_ = (em - jnp.int32(112 << 23) + jnp.int32(0x0FFF)
                                      + ((em >> 13) & 1)) >> 13
                                r_ = jnp.minimum(jnp.maximum(r_, 0),
                                                 jnp.int32(0x7BFF))
                                return s | r_

                            def conv_row(r, _3):
                                for g2 in range(dh // (2 * LANES)):
                                    a = narrow_f16(scaled0[r, pl.ds(g2 * 2 * LANES, LANES)])
                                    b = narrow_f16(scaled0[r, pl.ds(g2 * 2 * LANES + LANES, LANES)])
                                    rows0[r, pl.ds(g2 * LANES, LANES)] = a | (b << 16)
                                return 0

                            lax.fori_loop(0, nrows, conv_row, 0, unroll=False)
                            pltpu.sync_copy(rows0.at[pl.ds(0, nrows)],
                                            tb.at[pl.ds(poff, nrows)])

                        def conv_piece(p, _2, tb=tbs[l2]):
                            conv_span(pl.multiple_of(my_off + p * CHUNK, 8), CHUNK, tb)
                            return 0

                        lax.fori_loop(0, rows_per_tile // CHUNK, conv_piece, 0,
                                      unroll=False)
                        rem = rows_per_tile % CHUNK
                        if rem:
                            conv_span(
                                pl.multiple_of(
                                    my_off + (rows_per_tile // CHUNK) * CHUNK, 8),
                                rem, tbs[l2])
                plsc.subcore_barrier()
                return 0

            lax.fori_loop(0, num_layers, layer_body, 0, unroll=False)

        @pl.when(cid == 0)
        def _():
            run_layers(xa_hbm, [outs[2 * l] for l in range(num_layers)],
                       [tbls[2 * l] for l in range(num_layers - 1)])

        @pl.when(cid == 1)
        def _():
            run_layers(xb_hbm, [outs[2 * l + 1] for l in range(num_layers)],
                       [tbls[2 * l + 1] for l in range(num_layers - 1)])

    mesh = plsc.VectorSubcoreMesh(core_axis_name="c", subcore_axis_name="s")
    fn = pl.kernel(
        body,
        out_type=([jax.ShapeDtypeStruct((n_pad, dh), jnp.float32)] * nf32
                  + [jax.ShapeDtypeStruct((n_pad, dh // 2), jnp.int32)] * nbf),
        mesh=mesh,
        compiler_params=pltpu.CompilerParams(use_tc_tiling_on_sc=False,
                                             needs_layout_passes=False),
        scratch_types=[
            pltpu.VMEM((chunks, CHUNK), jnp.int32),
            pltpu.VMEM((chunks * CHUNK,), jnp.int32),
            pltpu.VMEM((chunks * CHUNK,), jnp.float32),
            pltpu.VMEM((CHUNK,), jnp.int32),
            pltpu.VMEM((CHUNK,), jnp.int32),
            pltpu.VMEM((CHUNK, dh // 2), jnp.int32),
            pltpu.VMEM((CHUNK, dh // 2), jnp.int32),
            pltpu.VMEM((CHUNK, dh), jnp.float32),
            pltpu.VMEM((CHUNK, dh), jnp.float32),
            pltpu.VMEM_SHARED((n_pad, dh), jnp.float32),
            pltpu.SemaphoreType.DMA,
            pltpu.SemaphoreType.DMA,
            pltpu.SemaphoreType.DMA,
            pltpu.SemaphoreType.DMA,
        ],
    )
    return fn(xa, xb, src2, dst2, w2, zrows)[:nf32]


def _mlp_body(xa, xb, h1a, h1b, h2a, h2b, h3a, h3b,
              w1_ref, b1_ref, w2_ref, b2_ref, out_ref):
    dh = xa.shape[1]
    parts = (xa, xb, h1a, h1b, h2a, h2b, h3a, h3b)
    acc = b1_ref[...].astype(jnp.float32)
    for k, p in enumerate(parts):
        acc = acc + jnp.dot(p[...], w1_ref[k * dh:(k + 1) * dh, :],
                            preferred_element_type=jnp.float32)
    hmid = jnp.tanh(acc)
    out_ref[...] = jnp.dot(hmid, w2_ref[...], preferred_element_type=jnp.float32) + b2_ref[...]


def _mlp_tc(parts, W1, b1, W2, b2, n, block_rows=1000):
    d = W2.shape[0]
    dh = parts[0].shape[1]
    grid = (n // block_rows,)
    row_spec = pl.BlockSpec((block_rows, dh), lambda i: (i, 0))
    full = lambda shape: pl.BlockSpec(shape, lambda i: tuple(0 for _ in shape))
    return pl.pallas_call(
        _mlp_body,
        grid=grid,
        in_specs=[row_spec] * 8 + [
            full(W1.shape), full((1, d)), full(W2.shape), full((1, d)),
        ],
        out_specs=pl.BlockSpec((block_rows, d), lambda i: (i, 0)),
        out_shape=jax.ShapeDtypeStruct((n, d), jnp.float32),
    )(*parts, W1, b1.reshape(1, d), W2, b2.reshape(1, d))


def kernel(x, edge_index, edge_weight, W1, b1, W2, b2):
    n, d = x.shape
    e = edge_index.shape[1]
    per_tile = e // NUM_SUBCORES
    chunks = per_tile // CHUNK
    num_layers = (W1.shape[0] // d) - 1
    dh = d // NUM_CORES

    align = 8 * NUM_SUBCORES
    n_pad = ((n + align - 1) // align) * align

    xa = x[:, :dh]
    xb = x[:, dh:]

    def _perm_bf16(m):
        n0 = m.shape[0]
        u = lax.bitcast_convert_type(m.astype(jnp.float16), jnp.uint16)
        u = (u.reshape(n0, dh // (2 * LANES), 2, LANES)
             .transpose(0, 1, 3, 2))
        return lax.bitcast_convert_type(u, jnp.int32).reshape(n0, dh // 2)

    src2 = edge_index[0].reshape(NUM_SUBCORES, chunks, CHUNK)
    dst2 = edge_index[1].reshape(NUM_SUBCORES, per_tile)
    w2 = edge_weight.reshape(NUM_SUBCORES, per_tile)
    zrows = jnp.zeros((n_pad // NUM_SUBCORES, dh), dtype=jnp.float32)

    hs = _gcn_sc(_perm_bf16(xa), _perm_bf16(xb), src2, dst2, w2, zrows,
                 n_pad, dh, chunks, num_layers)
    parts = [xa, xb] + [h[:n] for h in hs]
    return _mlp_tc(parts, W1, b1, W2, b2, n)


# speedup vs baseline: 1.7690x; 1.1670x over previous
import jax
import jax.numpy as jnp
from jax import lax
from jax.experimental import pallas as pl
from jax.experimental.pallas import tpu as pltpu
from jax.experimental.pallas import tpu_sc as plsc

NUM_CORES = 2
NUM_SUBCORES = 16
LANES = 16

CHUNK = 80


def _gcn_sc(xa, xb, src2, dst2, w2, zrows, n_pad, dh, chunks, num_layers):
    rows_per_tile = n_pad // NUM_SUBCORES
    groups = CHUNK // LANES
    nf32 = 2 * num_layers
    nbf = 2 * (num_layers - 1)

    def body(xa_hbm, xb_hbm, src_hbm, dst_hbm, w_hbm, zrows_hbm, *rest):
        outs = rest[:nf32]
        tbls = rest[nf32:nf32 + nbf]
        (srcb, dstb, wb, dstl0, dstl1, rows0, rows1, scaled0, scaled1, acc,
         sem_g0, sem_g1, sem_s0, sem_s1) = rest[nf32 + nbf:]
        cid = lax.axis_index("c")
        tid = lax.axis_index("s")

        pltpu.sync_copy(src_hbm.at[tid], srcb)
        pltpu.sync_copy(dst_hbm.at[tid], dstb)
        pltpu.sync_copy(w_hbm.at[tid], wb)

        my_off = pl.multiple_of(tid * rows_per_tile, 8)

        def scale_chunk(rows_r, scaled_r, dstl_r, i):
            @plsc.parallel_loop(0, groups, 1, unroll=5)
            def _(g):
                off = pl.multiple_of(i * CHUNK + g * LANES, LANES)
                wv16 = wb[pl.ds(off, LANES)]
                dstl_r[pl.ds(g * LANES, LANES)] = dstb[pl.ds(off, LANES)]
                wmask = jnp.int32(0x8FFFE000 - (1 << 32))
                wv16c = wv16 * jnp.float32(2.0 ** 112)
                for l in range(LANES):
                    wv = jnp.full((LANES,), wv16c[l])
                    e = g * LANES + l
                    for g2 in range(dh // (2 * LANES)):
                        v = rows_r[e, pl.ds(g2 * LANES, LANES)]
                        lo = plsc.bitcast(((v << 16) >> 3) & wmask, jnp.float32)
                        hi = plsc.bitcast((v >> 3) & wmask, jnp.float32)
                        scaled_r[e, pl.ds(g2 * 2 * LANES, LANES)] = lo * wv
                        scaled_r[e, pl.ds(g2 * 2 * LANES + LANES, LANES)] = hi * wv

        def run_layers(x_tab, houts, tbs):
            def layer_body(lay, _):
                tabs = [x_tab] + list(tbs)

                def gather_into(i, rows_r, sem):
                    for l2, tab in enumerate(tabs):
                        @pl.when(lay == l2)
                        def _(tab=tab):
                            pltpu.async_copy(tab.at[srcb.at[i]], rows_r, sem)

                def wait_gather(i, rows_r, sem):
                    for l2, tab in enumerate(tabs):
                        @pl.when(lay == l2)
                        def _(tab=tab):
                            pltpu.make_async_copy(tab.at[srcb.at[i]], rows_r, sem).wait()

                pltpu.sync_copy(zrows_hbm, acc.at[pl.ds(my_off, rows_per_tile)])
                gather_into(0, rows0, sem_g0)
                gather_into(1, rows1, sem_g1)
                plsc.subcore_barrier()

                def half_step(j, i, rows_r, scaled_r, dstl_r, sem_g, sem_s):
                    wait_gather(i, rows_r, sem_g)

                    @pl.when(j > 0)
                    def _():
                        pltpu.make_async_copy(scaled_r, acc.at[dstl_r], sem_s).wait()

                    scale_chunk(rows_r, scaled_r, dstl_r, i)

                    @pl.when(i + 2 < chunks)
                    def _():
                        gather_into(i + 2, rows_r, sem_g)

                    pltpu.async_copy(scaled_r, acc.at[dstl_r], sem_s, add=True)

                def pair_body(j, _):
                    half_step(j, 2 * j, rows0, scaled0, dstl0, sem_g0, sem_s0)
                    half_step(j, 2 * j + 1, rows1, scaled1, dstl1, sem_g1, sem_s1)
                    return 0

                lax.fori_loop(0, chunks // 2, pair_body, 0, unroll=False)
                pltpu.make_async_copy(scaled0, acc.at[dstl0], sem_s0).wait()
                pltpu.make_async_copy(scaled1, acc.at[dstl1], sem_s1).wait()
                plsc.subcore_barrier()

                for l2, hout in enumerate(houts):
                    @pl.when(lay == l2)
                    def _(l2=l2, hout=hout):
                        pltpu.sync_copy(acc.at[pl.ds(my_off, rows_per_tile)],
                                        hout.at[pl.ds(my_off, rows_per_tile)])
                        if l2 >= num_layers - 1:
                            return

                        def conv_span(poff, nrows, tb):
                            pltpu.sync_copy(acc.at[pl.ds(poff, nrows)],
                                            scaled0.at[pl.ds(0, nrows)])

                            def narrow_f16(f):
                                bb = plsc.bitcast(f, jnp.int32)
                                s = (bb >> 16) & jnp.int32(0x8000)
                                em = bb & jnp.int32(0x7FFFFFFF)
                                r_ = (em - jnp.int32(112 << 23) + jnp.int32(0x0FFF)
                                      + ((em >> 13) & 1)) >> 13
                                r_ = jnp.minimum(jnp.maximum(r_, 0),
                                                 jnp.int32(0x7BFF))
                                return s | r_

                            def conv_row(r, _3):
                                for g2 in range(dh // (2 * LANES)):
                                    a = narrow_f16(scaled0[r, pl.ds(g2 * 2 * LANES, LANES)])
                                    b = narrow_f16(scaled0[r, pl.ds(g2 * 2 * LANES + LANES, LANES)])
                                    rows0[r, pl.ds(g2 * LANES, LANES)] = a | (b << 16)
                                return 0

                            lax.fori_loop(0, nrows, conv_row, 0, unroll=False)
                            pltpu.sync_copy(rows0.at[pl.ds(0, nrows)],
                                            tb.at[pl.ds(poff, nrows)])

                        def conv_piece(p, _2, tb=tbs[l2]):
                            conv_span(pl.multiple_of(my_off + p * CHUNK, 8), CHUNK, tb)
                            return 0

                        lax.fori_loop(0, rows_per_tile // CHUNK, conv_piece, 0,
                                      unroll=False)
                        rem = rows_per_tile % CHUNK
                        if rem:
                            conv_span(
                                pl.multiple_of(
                                    my_off + (rows_per_tile // CHUNK) * CHUNK, 8),
                                rem, tbs[l2])
                plsc.subcore_barrier()
                return 0

            lax.fori_loop(0, num_layers, layer_body, 0, unroll=False)

        @pl.when(cid == 0)
        def _():
            run_layers(xa_hbm, [outs[2 * l] for l in range(num_layers)],
                       [tbls[2 * l] for l in range(num_layers - 1)])

        @pl.when(cid == 1)
        def _():
            run_layers(xb_hbm, [outs[2 * l + 1] for l in range(num_layers)],
                       [tbls[2 * l + 1] for l in range(num_layers - 1)])

    mesh = plsc.VectorSubcoreMesh(core_axis_name="c", subcore_axis_name="s")
    fn = pl.kernel(
        body,
        out_type=([jax.ShapeDtypeStruct((n_pad, dh), jnp.float32)] * nf32
                  + [jax.ShapeDtypeStruct((n_pad, dh // 2), jnp.int32)] * nbf),
        mesh=mesh,
        compiler_params=pltpu.CompilerParams(use_tc_tiling_on_sc=False,
                                             needs_layout_passes=False),
        scratch_types=[
            pltpu.VMEM((chunks, CHUNK), jnp.int32),
            pltpu.VMEM((chunks * CHUNK,), jnp.int32),
            pltpu.VMEM((chunks * CHUNK,), jnp.float32),
            pltpu.VMEM((CHUNK,), jnp.int32),
            pltpu.VMEM((CHUNK,), jnp.int32),
            pltpu.VMEM((CHUNK, dh // 2), jnp.int32),
            pltpu.VMEM((CHUNK, dh // 2), jnp.int32),
            pltpu.VMEM((CHUNK, dh), jnp.float32),
            pltpu.VMEM((CHUNK, dh), jnp.float32),
            pltpu.VMEM_SHARED((n_pad, dh), jnp.float32),
            pltpu.SemaphoreType.DMA,
            pltpu.SemaphoreType.DMA,
            pltpu.SemaphoreType.DMA,
            pltpu.SemaphoreType.DMA,
        ],
    )
    return fn(xa, xb, src2, dst2, w2, zrows)[:nf32]


def _mlp_body(xa, xb, h1a, h1b, h2a, h2b, h3a, h3b,
              w1_ref, b1_ref, w2_ref, b2_ref, out_ref):
    dh = xa.shape[1]
    parts = (xa, xb, h1a, h1b, h2a, h2b, h3a, h3b)
    acc = b1_ref[...].astype(jnp.float32)
    for k, p in enumerate(parts):
        acc = acc + jnp.dot(p[...], w1_ref[k * dh:(k + 1) * dh, :],
                            preferred_element_type=jnp.float32)
    hmid = jnp.tanh(acc)
    out_ref[...] = jnp.dot(hmid, w2_ref[...], preferred_element_type=jnp.float32) + b2_ref[...]


def _mlp_tc(parts, W1, b1, W2, b2, n, block_rows=1000):
    d = W2.shape[0]
    dh = parts[0].shape[1]
    grid = (n // block_rows,)
    row_spec = pl.BlockSpec((block_rows, dh), lambda i: (i, 0))
    full = lambda shape: pl.BlockSpec(shape, lambda i: tuple(0 for _ in shape))
    return pl.pallas_call(
        _mlp_body,
        grid=grid,
        in_specs=[row_spec] * 8 + [
            full(W1.shape), full((1, d)), full(W2.shape), full((1, d)),
        ],
        out_specs=pl.BlockSpec((block_rows, d), lambda i: (i, 0)),
        out_shape=jax.ShapeDtypeStruct((n, d), jnp.float32),
    )(*parts, W1, b1.reshape(1, d), W2, b2.reshape(1, d))


def kernel(x, edge_index, edge_weight, W1, b1, W2, b2):
    n, d = x.shape
    e = edge_index.shape[1]
    per_tile = e // NUM_SUBCORES
    chunks = per_tile // CHUNK
    num_layers = (W1.shape[0] // d) - 1
    dh = d // NUM_CORES

    align = 8 * NUM_SUBCORES
    n_pad = ((n + align - 1) // align) * align

    xa = x[:, :dh]
    xb = x[:, dh:]

    def _perm_bf16(m):
        n0 = m.shape[0]
        u = lax.bitcast_convert_type(m.astype(jnp.float16), jnp.uint16)
        u = (u.reshape(n0, dh // (2 * LANES), 2, LANES)
             .transpose(0, 1, 3, 2))
        return lax.bitcast_convert_type(u, jnp.int32).reshape(n0, dh // 2)

    src2 = edge_index[0].reshape(NUM_SUBCORES, chunks, CHUNK)
    dst2 = edge_index[1].reshape(NUM_SUBCORES, per_tile)
    w2 = edge_weight.reshape(NUM_SUBCORES, per_tile)
    zrows = jnp.zeros((n_pad // NUM_SUBCORES, dh), dtype=jnp.float32)

    hs = _gcn_sc(_perm_bf16(xa), _perm_bf16(xb), src2, dst2, w2, zrows,
                 n_pad, dh, chunks, num_layers)
    parts = [xa, xb] + [h[:n] for h in hs]
    return _mlp_tc(parts, W1, b1, W2, b2, n)
